# baseline (speedup 1.0000x reference)
import functools

import jax
import jax.numpy as jnp
from jax import lax
from jax.experimental import pallas as pl
from jax.experimental.pallas import tpu as pltpu
from jax.experimental.pallas import tpu_sc as plsc

B, T, D, E, H, K = 1, 2048, 768, 16, 64, 2
SH = H * K
N = B * T
BT = 512

_NC, _NS = 2, 16
_NW = _NC * _NS
_TPW = N // _NW


def _logits_block(x_ref, rw_ref, bias_ref, out_ref):
    logits = jax.lax.dot_general(x_ref[...], rw_ref[...],
                                 (((1,), (1,)), ((), ())),
                                 preferred_element_type=jnp.float32)
    out_ref[...] = logits + bias_ref[...]


def _bfly_max(x, xs):
    for s in (8, 4, 2, 1):
        x = jnp.maximum(x, x.at[xs[s]].get(mode="promise_in_bounds"))
    return x


def _router_sc(logits_hbm, w_hbm, buf, wout):
    wid = lax.axis_index("s") * _NC + lax.axis_index("c")
    base = wid * (_TPW * E)
    pltpu.sync_copy(logits_hbm.at[pl.ds(base, _TPW * E)], buf)
    col = jax.lax.broadcasted_iota(jnp.int32, (16,), 0)
    xs = {s: jnp.bitwise_xor(col, s) for s in (8, 4, 2, 1)}
    for t in range(_TPW):
        v = buf[pl.ds(t * E, E)]
        m1v = _bfly_max(v, xs)
        i1v = 15 - _bfly_max(jnp.where(v == m1v, 15 - col, -1), xs)
        masked = jnp.where(col == i1v, jnp.float32(-3.0e38), v)
        m2v = _bfly_max(masked, xs)
        i2v = 15 - _bfly_max(jnp.where(masked == m2v, 15 - col, -1), xs)
        e2 = jnp.exp(m2v - m1v)
        w1 = 1.0 / (1.0 + e2)
        w2 = e2 * w1
        w = jnp.where(col == i1v, w1, jnp.where(col == i2v, w2, 0.0))
        wout[pl.ds(t * E, E)] = w
    pltpu.sync_copy(wout, w_hbm.at[pl.ds(base, _TPW * E)])


def _router_jit(logits_flat):
    router = functools.partial(
        pl.kernel,
        out_type=jax.ShapeDtypeStruct((N * E,), jnp.float32),
        mesh=plsc.VectorSubcoreMesh(core_axis_name="c", subcore_axis_name="s"),
        scratch_types=[
            pltpu.VMEM((_TPW * E,), jnp.float32),
            pltpu.VMEM((_TPW * E,), jnp.float32),
        ],
    )(_router_sc)
    return router(logits_flat)


def _moe_block(x_ref, w_ref, up_ref, gate_ref, down_ref,
               sg_ref, su_ref, sd_ref, out_ref, up_s, gate_s, down_s):
    f32 = jnp.float32
    bf = jnp.bfloat16

    @pl.when(pl.program_id(0) == 0)
    def _build_fused():
        up_s[...] = jnp.concatenate([up_ref[e] for e in range(E)], axis=1)
        gate_s[...] = jnp.concatenate([gate_ref[e] for e in range(E)], axis=1)
        down_s[...] = jnp.concatenate([down_ref[e] for e in range(E)], axis=0)

    x = x_ref[...]
    rep = (jax.lax.broadcasted_iota(jnp.int32, (E, E * H), 1) // H ==
           jax.lax.broadcasted_iota(jnp.int32, (E, E * H), 0)).astype(f32)
    wexp = jnp.dot(w_ref[...], rep, preferred_element_type=f32)
    u = jnp.dot(x, up_s[...], preferred_element_type=f32)
    g = jnp.dot(x, gate_s[...], preferred_element_type=f32)
    h = (g * jax.nn.sigmoid(g)) * u * wexp
    acc = jnp.dot(h, down_s[...], preferred_element_type=f32)
    sg = jax.lax.dot_general(x, sg_ref[...], (((1,), (1,)), ((), ())),
                             preferred_element_type=f32)
    su = jax.lax.dot_general(x, su_ref[...], (((1,), (1,)), ((), ())),
                             preferred_element_type=f32)
    sh = (sg * jax.nn.sigmoid(sg)) * su
    acc = acc + jax.lax.dot_general(sh, sd_ref[...], (((1,), (1,)), ((), ())),
                                    preferred_element_type=f32)
    out_ref[...] = acc


@jax.jit
def _moe(flat, rw, bias2, up_w, gate_w, down_w, sg_w, su_w, sd_w):
    logits = pl.pallas_call(
        _logits_block,
        in_specs=[
            pl.BlockSpec((N, D), lambda: (0, 0)),
            pl.BlockSpec((E, D), lambda: (0, 0)),
            pl.BlockSpec((1, E), lambda: (0, 0)),
        ],
        out_specs=pl.BlockSpec((N, E), lambda: (0, 0)),
        out_shape=jax.ShapeDtypeStruct((N, E), jnp.float32),
    )(flat, rw, bias2)

    w = _router_jit(logits.reshape(N * E)).reshape(N, E)

    grid = (N // BT,)
    full2 = lambda i: (0, 0)
    full3 = lambda i: (0, 0, 0)
    return pl.pallas_call(
        _moe_block,
        grid=grid,
        in_specs=[
            pl.BlockSpec((BT, D), lambda i: (i, 0)),
            pl.BlockSpec((BT, E), lambda i: (i, 0)),
            pl.BlockSpec((E, D, H), full3),
            pl.BlockSpec((E, D, H), full3),
            pl.BlockSpec((E, H, D), full3),
            pl.BlockSpec((SH, D), full2),
            pl.BlockSpec((SH, D), full2),
            pl.BlockSpec((D, SH), full2),
        ],
        out_specs=pl.BlockSpec((BT, D), lambda i: (i, 0)),
        out_shape=jax.ShapeDtypeStruct((N, D), jnp.float32),
        scratch_shapes=[
            pltpu.VMEM((D, E * H), jnp.float32),
            pltpu.VMEM((D, E * H), jnp.float32),
            pltpu.VMEM((E * H, D), jnp.float32),
        ],
    )(flat, w, up_w, gate_w, down_w, sg_w, su_w, sd_w)


def kernel(x, router_w, router_bias, up_proj, gate_proj, down_proj,
           shared_gate_w, shared_up_w, shared_down_w):
    flat = x.reshape(N, D)
    bias2 = router_bias.reshape(1, E)
    out = _moe(flat, router_w, bias2, up_proj, gate_proj, down_proj,
               shared_gate_w, shared_up_w, shared_down_w)
    return out.reshape(B, T, D)


# speedup vs baseline: 1.0300x; 1.0300x over previous
import functools

import jax
import jax.numpy as jnp
from jax import lax
from jax.experimental import pallas as pl
from jax.experimental.pallas import tpu as pltpu
from jax.experimental.pallas import tpu_sc as plsc

B, T, D, E, H, K = 1, 2048, 768, 16, 64, 2
SH = H * K
N = B * T
BT = 1024

_NC, _NS = 2, 16
_NW = _NC * _NS
_TPW = N // _NW


def _logits_block(x_ref, rw_ref, bias_ref, out_ref):
    logits = jax.lax.dot_general(x_ref[...], rw_ref[...],
                                 (((1,), (1,)), ((), ())),
                                 preferred_element_type=jnp.float32)
    out_ref[...] = logits + bias_ref[...]


def _bfly_max(x, xs):
    for s in (8, 4, 2, 1):
        x = jnp.maximum(x, x.at[xs[s]].get(mode="promise_in_bounds"))
    return x


def _router_sc(logits_hbm, w_hbm, buf, wout):
    wid = lax.axis_index("s") * _NC + lax.axis_index("c")
    base = wid * (_TPW * E)
    pltpu.sync_copy(logits_hbm.at[pl.ds(base, _TPW * E)], buf)
    col = jax.lax.broadcasted_iota(jnp.int32, (16,), 0)
    xs = {s: jnp.bitwise_xor(col, s) for s in (8, 4, 2, 1)}
    for t in range(_TPW):
        v = buf[pl.ds(t * E, E)]
        m1v = _bfly_max(v, xs)
        i1v = 15 - _bfly_max(jnp.where(v == m1v, 15 - col, -1), xs)
        masked = jnp.where(col == i1v, jnp.float32(-3.0e38), v)
        m2v = _bfly_max(masked, xs)
        i2v = 15 - _bfly_max(jnp.where(masked == m2v, 15 - col, -1), xs)
        e2 = jnp.exp(m2v - m1v)
        w1 = 1.0 / (1.0 + e2)
        w2 = e2 * w1
        w = jnp.where(col == i1v, w1, jnp.where(col == i2v, w2, 0.0))
        wout[pl.ds(t * E, E)] = w
    pltpu.sync_copy(wout, w_hbm.at[pl.ds(base, _TPW * E)])


def _router_jit(logits_flat):
    router = functools.partial(
        pl.kernel,
        out_type=jax.ShapeDtypeStruct((N * E,), jnp.float32),
        mesh=plsc.VectorSubcoreMesh(core_axis_name="c", subcore_axis_name="s"),
        scratch_types=[
            pltpu.VMEM((_TPW * E,), jnp.float32),
            pltpu.VMEM((_TPW * E,), jnp.float32),
        ],
    )(_router_sc)
    return router(logits_flat)


def _moe_block(x_ref, w_ref, up_ref, gate_ref, down_ref,
               sg_ref, su_ref, sd_ref, out_ref, up_s, gate_s, down_s):
    f32 = jnp.float32
    bf = jnp.bfloat16

    @pl.when(pl.program_id(0) == 0)
    def _build_fused():
        up_s[...] = jnp.concatenate(
            [up_ref[e] for e in range(E)], axis=1).astype(bf)
        gate_s[...] = jnp.concatenate(
            [gate_ref[e] for e in range(E)], axis=1).astype(bf)
        down_s[...] = jnp.concatenate(
            [down_ref[e] for e in range(E)], axis=0).astype(bf)

    x = x_ref[...]
    rep = (jax.lax.broadcasted_iota(jnp.int32, (E, E * H), 1) // H ==
           jax.lax.broadcasted_iota(jnp.int32, (E, E * H), 0)).astype(f32)
    wexp = jnp.dot(w_ref[...], rep, preferred_element_type=f32)
    xb = x.astype(bf)
    u = jnp.dot(xb, up_s[...], preferred_element_type=f32)
    g = jnp.dot(xb, gate_s[...], preferred_element_type=f32)
    h = (g * jax.nn.sigmoid(g)) * u * wexp
    acc = jnp.dot(h.astype(bf), down_s[...], preferred_element_type=f32)
    sg = jax.lax.dot_general(x, sg_ref[...], (((1,), (1,)), ((), ())),
                             preferred_element_type=f32)
    su = jax.lax.dot_general(x, su_ref[...], (((1,), (1,)), ((), ())),
                             preferred_element_type=f32)
    sh = (sg * jax.nn.sigmoid(sg)) * su
    acc = acc + jax.lax.dot_general(sh, sd_ref[...], (((1,), (1,)), ((), ())),
                                    preferred_element_type=f32)
    out_ref[...] = acc


@jax.jit
def _moe(flat, rw, bias2, up_w, gate_w, down_w, sg_w, su_w, sd_w):
    logits = pl.pallas_call(
        _logits_block,
        in_specs=[
            pl.BlockSpec((N, D), lambda: (0, 0)),
            pl.BlockSpec((E, D), lambda: (0, 0)),
            pl.BlockSpec((1, E), lambda: (0, 0)),
        ],
        out_specs=pl.BlockSpec((N, E), lambda: (0, 0)),
        out_shape=jax.ShapeDtypeStruct((N, E), jnp.float32),
    )(flat, rw, bias2)

    w = _router_jit(logits.reshape(N * E)).reshape(N, E)

    grid = (N // BT,)
    full2 = lambda i: (0, 0)
    full3 = lambda i: (0, 0, 0)
    return pl.pallas_call(
        _moe_block,
        grid=grid,
        in_specs=[
            pl.BlockSpec((BT, D), lambda i: (i, 0)),
            pl.BlockSpec((BT, E), lambda i: (i, 0)),
            pl.BlockSpec((E, D, H), full3),
            pl.BlockSpec((E, D, H), full3),
            pl.BlockSpec((E, H, D), full3),
            pl.BlockSpec((SH, D), full2),
            pl.BlockSpec((SH, D), full2),
            pl.BlockSpec((D, SH), full2),
        ],
        out_specs=pl.BlockSpec((BT, D), lambda i: (i, 0)),
        out_shape=jax.ShapeDtypeStruct((N, D), jnp.float32),
        scratch_shapes=[
            pltpu.VMEM((D, E * H), jnp.bfloat16),
            pltpu.VMEM((D, E * H), jnp.bfloat16),
            pltpu.VMEM((E * H, D), jnp.bfloat16),
        ],
    )(flat, w, up_w, gate_w, down_w, sg_w, su_w, sd_w)


def kernel(x, router_w, router_bias, up_proj, gate_proj, down_proj,
           shared_gate_w, shared_up_w, shared_down_w):
    flat = x.reshape(N, D)
    bias2 = router_bias.reshape(1, E)
    out = _moe(flat, router_w, bias2, up_proj, gate_proj, down_proj,
               shared_gate_w, shared_up_w, shared_down_w)
    return out.reshape(B, T, D)


# speedup vs baseline: 1.0365x; 1.0063x over previous
import functools

import jax
import jax.numpy as jnp
from jax import lax
from jax.experimental import pallas as pl
from jax.experimental.pallas import tpu as pltpu
from jax.experimental.pallas import tpu_sc as plsc

B, T, D, E, H, K = 1, 2048, 768, 16, 64, 2
SH = H * K
N = B * T
BT = 1024

_NC, _NS = 2, 16
_NW = _NC * _NS
_TPW = N // _NW


_BL = 512


def _logits_block(x_ref, rw_ref, bias_ref, out_ref):
    logits = jax.lax.dot_general(x_ref[...], rw_ref[...],
                                 (((1,), (1,)), ((), ())),
                                 preferred_element_type=jnp.float32)
    out_ref[...] = logits + bias_ref[...]


def _bfly_max(x, xs):
    for s in (8, 4, 2, 1):
        x = jnp.maximum(x, x.at[xs[s]].get(mode="promise_in_bounds"))
    return x


def _router_sc(logits_hbm, w_hbm, buf, wout):
    wid = lax.axis_index("s") * _NC + lax.axis_index("c")
    base = wid * (_TPW * E)
    pltpu.sync_copy(logits_hbm.at[pl.ds(base, _TPW * E)], buf)
    col = jax.lax.broadcasted_iota(jnp.int32, (16,), 0)
    xs = {s: jnp.bitwise_xor(col, s) for s in (8, 4, 2, 1)}
    for t in range(_TPW):
        v = buf[pl.ds(t * E, E)]
        m1v = _bfly_max(v, xs)
        i1v = 15 - _bfly_max(jnp.where(v == m1v, 15 - col, -1), xs)
        masked = jnp.where(col == i1v, jnp.float32(-3.0e38), v)
        m2v = _bfly_max(masked, xs)
        i2v = 15 - _bfly_max(jnp.where(masked == m2v, 15 - col, -1), xs)
        e2 = jnp.exp(m2v - m1v)
        w1 = 1.0 / (1.0 + e2)
        w2 = e2 * w1
        w = jnp.where(col == i1v, w1, jnp.where(col == i2v, w2, 0.0))
        wout[pl.ds(t * E, E)] = w
    pltpu.sync_copy(wout, w_hbm.at[pl.ds(base, _TPW * E)])


def _router_jit(logits_flat):
    router = functools.partial(
        pl.kernel,
        out_type=jax.ShapeDtypeStruct((N * E,), jnp.float32),
        mesh=plsc.VectorSubcoreMesh(core_axis_name="c", subcore_axis_name="s"),
        scratch_types=[
            pltpu.VMEM((_TPW * E,), jnp.float32),
            pltpu.VMEM((_TPW * E,), jnp.float32),
        ],
    )(_router_sc)
    return router(logits_flat)


def _moe_block(x_ref, w_ref, up_ref, gate_ref, down_ref,
               sg_ref, su_ref, sd_ref, out_ref, up_s, gate_s, down_s):
    f32 = jnp.float32
    bf = jnp.bfloat16

    @pl.when(pl.program_id(0) == 0)
    def _build_fused():
        up_s[...] = jnp.concatenate(
            [up_ref[e] for e in range(E)], axis=1).astype(bf)
        gate_s[...] = jnp.concatenate(
            [gate_ref[e] for e in range(E)], axis=1).astype(bf)
        down_s[...] = jnp.concatenate(
            [down_ref[e] for e in range(E)], axis=0).astype(bf)

    x = x_ref[...]
    rep = (jax.lax.broadcasted_iota(jnp.int32, (E, E * H), 1) // H ==
           jax.lax.broadcasted_iota(jnp.int32, (E, E * H), 0)).astype(f32)
    wexp = jnp.dot(w_ref[...], rep, preferred_element_type=f32)
    xb = x.astype(bf)
    u = jnp.dot(xb, up_s[...], preferred_element_type=f32)
    g = jnp.dot(xb, gate_s[...], preferred_element_type=f32)
    h = (g * jax.nn.sigmoid(g)) * u * wexp
    acc = jnp.dot(h.astype(bf), down_s[...], preferred_element_type=f32)
    sg = jax.lax.dot_general(x, sg_ref[...], (((1,), (1,)), ((), ())),
                             preferred_element_type=f32)
    su = jax.lax.dot_general(x, su_ref[...], (((1,), (1,)), ((), ())),
                             preferred_element_type=f32)
    sh = (sg * jax.nn.sigmoid(sg)) * su
    acc = acc + jax.lax.dot_general(sh, sd_ref[...], (((1,), (1,)), ((), ())),
                                    preferred_element_type=f32)
    out_ref[...] = acc


@jax.jit
def _moe(flat, rw, bias2, up_w, gate_w, down_w, sg_w, su_w, sd_w):
    logits = pl.pallas_call(
        _logits_block,
        grid=(N // _BL,),
        in_specs=[
            pl.BlockSpec((_BL, D), lambda i: (i, 0)),
            pl.BlockSpec((E, D), lambda i: (0, 0)),
            pl.BlockSpec((1, E), lambda i: (0, 0)),
        ],
        out_specs=pl.BlockSpec((_BL, E), lambda i: (i, 0)),
        out_shape=jax.ShapeDtypeStruct((N, E), jnp.float32),
    )(flat, rw, bias2)

    w = _router_jit(logits.reshape(N * E)).reshape(N, E)

    grid = (N // BT,)
    full2 = lambda i: (0, 0)
    full3 = lambda i: (0, 0, 0)
    return pl.pallas_call(
        _moe_block,
        grid=grid,
        in_specs=[
            pl.BlockSpec((BT, D), lambda i: (i, 0)),
            pl.BlockSpec((BT, E), lambda i: (i, 0)),
            pl.BlockSpec((E, D, H), full3),
            pl.BlockSpec((E, D, H), full3),
            pl.BlockSpec((E, H, D), full3),
            pl.BlockSpec((SH, D), full2),
            pl.BlockSpec((SH, D), full2),
            pl.BlockSpec((D, SH), full2),
        ],
        out_specs=pl.BlockSpec((BT, D), lambda i: (i, 0)),
        out_shape=jax.ShapeDtypeStruct((N, D), jnp.float32),
        scratch_shapes=[
            pltpu.VMEM((D, E * H), jnp.bfloat16),
            pltpu.VMEM((D, E * H), jnp.bfloat16),
            pltpu.VMEM((E * H, D), jnp.bfloat16),
        ],
    )(flat, w, up_w, gate_w, down_w, sg_w, su_w, sd_w)


def kernel(x, router_w, router_bias, up_proj, gate_proj, down_proj,
           shared_gate_w, shared_up_w, shared_down_w):
    flat = x.reshape(N, D)
    bias2 = router_bias.reshape(1, E)
    out = _moe(flat, router_w, bias2, up_proj, gate_proj, down_proj,
               shared_gate_w, shared_up_w, shared_down_w)
    return out.reshape(B, T, D)
